# Initial kernel scaffold; baseline (speedup 1.0000x reference)
import jax
import jax.numpy as jnp
from jax.experimental import pallas as pl


def kernel(x, edge_index, W1, b1, W2, b2, W3, b3, Wc, bc):
    raise NotImplementedError("write your pallas kernel here")


# speedup vs baseline: 68.4302x; 68.4302x over previous
import functools

import jax
import jax.numpy as jnp
from jax import lax
from jax.experimental import pallas as pl
from jax.experimental.pallas import tpu as pltpu
from jax.experimental.pallas import tpu_sc as plsc

_N = 100000
_E = 6400000
_F = 8
_LANES = 128
_GROUP = 16
_NW = 32
_ROWS_PER_W = 1568
_R = _NW * _ROWS_PER_W
_EPAD = _R * _LANES
_NPAD = 100096


def _build_kernels(npad, lanes, group, rows_per_w, f, interpret=False):
    npt = npad // 16
    mesh = plsc.VectorSubcoreMesh(core_axis_name="c", subcore_axis_name="s",
                                  num_cores=2, num_subcores=16)
    cparams = pltpu.CompilerParams(use_tc_tiling_on_sc=False)

    @functools.partial(
        pl.kernel,
        mesh=mesh,
        out_type=jax.ShapeDtypeStruct((2 * npad,), jnp.float32),
        scratch_types=[
            pltpu.VMEM((group, lanes), jnp.int32),
            pltpu.VMEM((lanes,), jnp.float32),
            pltpu.VMEM((npt,), jnp.float32),
            pltpu.VMEM_SHARED((npad,), jnp.float32),
        ],
        compiler_params=cparams,
        interpret=interpret,
    )
    def deg_kernel(dst_hbm, zeros_hbm, out_hbm, didx, ones_v, bnc,
                   acc_sh):
        c = lax.axis_index("c")
        s = lax.axis_index("s")
        wid = c * 16 + s

        for i in range(lanes // 16):
            ones_v[pl.ds(i * 16, 16)] = jnp.ones((16,), jnp.float32)
        pltpu.sync_copy(zeros_hbm.at[pl.ds(s * npt, npt)], bnc)
        pltpu.sync_copy(bnc, acc_sh.at[pl.ds(s * npt, npt)])
        plsc.subcore_barrier()

        base = wid * rows_per_w

        def body(g, carry):
            r0 = base + g * group
            pltpu.sync_copy(dst_hbm.at[pl.ds(r0, group)], didx)
            for j in range(group):
                pltpu.sync_copy(ones_v, acc_sh.at[didx.at[j]], add=True)
            return carry

        lax.fori_loop(0, rows_per_w // group, body, 0)
        plsc.subcore_barrier()
        pltpu.sync_copy(acc_sh.at[pl.ds(s * npt, npt)], bnc)
        pltpu.sync_copy(bnc, out_hbm.at[pl.ds(c * npad + s * npt, npt)])

    @functools.partial(
        pl.kernel,
        mesh=mesh,
        out_type=jax.ShapeDtypeStruct((2 * npad, f), jnp.float32),
        scratch_types=[
            pltpu.VMEM((group, lanes), jnp.int32),
            pltpu.VMEM((group, lanes), jnp.int32),
            pltpu.VMEM((group, lanes, f), jnp.float32),
            pltpu.VMEM((npt, f), jnp.float32),
            pltpu.VMEM_SHARED((npad, f), jnp.float32),
            pltpu.SemaphoreType.DMA,
            pltpu.SemaphoreType.DMA,
        ],
        compiler_params=cparams,
        interpret=interpret,
    )
    def edge_kernel(table_hbm, src_hbm, dst_hbm, zeros_hbm, out_hbm,
                    sidx, didx, rows, bnc, acc_sh, sem, sem2):
        c = lax.axis_index("c")
        s = lax.axis_index("s")
        wid = c * 16 + s

        pltpu.sync_copy(zeros_hbm.at[pl.ds(s * npt, npt)], bnc)
        pltpu.sync_copy(bnc, acc_sh.at[pl.ds(s * npt, npt)])
        plsc.subcore_barrier()

        base = wid * rows_per_w

        def body(g, carry):
            r0 = base + g * group
            pltpu.sync_copy(src_hbm.at[pl.ds(r0, group)], sidx)
            pltpu.sync_copy(dst_hbm.at[pl.ds(r0, group)], didx)
            handles = []
            for j in range(group):
                handles.append(
                    pltpu.async_copy(table_hbm.at[sidx.at[j]], rows.at[j],
                                     sem))
            for h in handles:
                h.wait()
            shandles = []
            for j in range(group):
                shandles.append(
                    pltpu.async_copy(rows.at[j], acc_sh.at[didx.at[j]], sem2,
                                     add=True))
            for h in shandles:
                h.wait()
            return carry

        lax.fori_loop(0, rows_per_w // group, body, 0)
        plsc.subcore_barrier()
        pltpu.sync_copy(acc_sh.at[pl.ds(s * npt, npt)], bnc)
        pltpu.sync_copy(bnc, out_hbm.at[pl.ds(c * npad + s * npt, npt)])

    return deg_kernel, edge_kernel


_deg_kernel, _edge_kernel = _build_kernels(_NPAD, _LANES, _GROUP, _ROWS_PER_W,
                                           _F)


def _gcn(deg_kernel, edge_kernel, n, npad, epad,
         x, edge_index, W1, b1, W2, b2, W3, b3, Wc, bc):
    e = edge_index.shape[1]
    lanes = _LANES
    pad_e = epad - e
    src = jnp.concatenate(
        [edge_index[0], jnp.full((pad_e,), n, jnp.int32)]).reshape(-1, lanes)
    dst = jnp.concatenate(
        [edge_index[1], jnp.full((pad_e,), n, jnp.int32)]).reshape(-1, lanes)

    zeros1 = jnp.zeros((npad,), jnp.float32)
    zerosF = jnp.zeros((npad, _F), jnp.float32)

    degp = deg_kernel(dst, zeros1)
    deg = degp[:n] + degp[npad:npad + n] + 1.0
    dis = lax.rsqrt(deg)

    def layer(h, W, b, fout):
        ts = (h @ W) * dis[:, None]
        tsp = jnp.zeros((npad, _F), jnp.float32).at[:n, :fout].set(ts)
        part = edge_kernel(tsp, src, dst, zerosF)
        agg = part[:n, :fout] + part[npad:npad + n, :fout] + ts
        return jnp.tanh(agg * dis[:, None] + b)

    h = layer(x, W1, b1, 4)
    h = layer(h, W2, b2, 4)
    h = layer(h, W3, b3, 3)
    out = h @ Wc + bc
    return (out, h)


def kernel(x, edge_index, W1, b1, W2, b2, W3, b3, Wc, bc):
    return _gcn(_deg_kernel, _edge_kernel, _N, _NPAD, _EPAD,
                x, edge_index, W1, b1, W2, b2, W3, b3, Wc, bc)


# speedup vs baseline: 74.6318x; 1.0906x over previous
import functools

import jax
import jax.numpy as jnp
from jax import lax
from jax.experimental import pallas as pl
from jax.experimental.pallas import tpu as pltpu
from jax.experimental.pallas import tpu_sc as plsc

_N = 100000
_E = 6400000
_F = 8
_LANES = 128
_GROUP = 16
_NW = 32
_ROWS_PER_W = 1568
_R = _NW * _ROWS_PER_W
_EPAD = _R * _LANES
_NPAD = 100096


def _build_kernels(npad, lanes, group, rows_per_w, f, interpret=False):
    npt = npad // 16
    mesh = plsc.VectorSubcoreMesh(core_axis_name="c", subcore_axis_name="s",
                                  num_cores=2, num_subcores=16)
    cparams = pltpu.CompilerParams(use_tc_tiling_on_sc=False)

    @functools.partial(
        pl.kernel,
        mesh=mesh,
        out_type=jax.ShapeDtypeStruct((2 * npad,), jnp.float32),
        scratch_types=[
            pltpu.VMEM((group, lanes), jnp.int32),
            pltpu.VMEM((lanes,), jnp.float32),
            pltpu.VMEM((npt,), jnp.float32),
            pltpu.VMEM_SHARED((npad,), jnp.float32),
        ],
        compiler_params=cparams,
        interpret=interpret,
    )
    def deg_kernel(dst_hbm, zeros_hbm, out_hbm, didx, ones_v, bnc,
                   acc_sh):
        c = lax.axis_index("c")
        s = lax.axis_index("s")
        wid = c * 16 + s

        for i in range(lanes // 16):
            ones_v[pl.ds(i * 16, 16)] = jnp.ones((16,), jnp.float32)
        pltpu.sync_copy(zeros_hbm.at[pl.ds(s * npt, npt)], bnc)
        pltpu.sync_copy(bnc, acc_sh.at[pl.ds(s * npt, npt)])
        plsc.subcore_barrier()

        base = wid * rows_per_w

        def body(g, carry):
            r0 = base + g * group
            pltpu.sync_copy(dst_hbm.at[pl.ds(r0, group)], didx)
            for j in range(group):
                pltpu.sync_copy(ones_v, acc_sh.at[didx.at[j]], add=True)
            return carry

        lax.fori_loop(0, rows_per_w // group, body, 0)
        plsc.subcore_barrier()
        pltpu.sync_copy(acc_sh.at[pl.ds(s * npt, npt)], bnc)
        pltpu.sync_copy(bnc, out_hbm.at[pl.ds(c * npad + s * npt, npt)])

    @functools.partial(
        pl.kernel,
        mesh=mesh,
        out_type=jax.ShapeDtypeStruct((2 * npad, f), jnp.float32),
        scratch_types=[
            pltpu.VMEM((2 * group, lanes), jnp.int32),
            pltpu.VMEM((2 * group, lanes), jnp.int32),
            pltpu.VMEM((2 * group, lanes, f), jnp.float32),
            pltpu.VMEM((npt // 4, f), jnp.float32),
            pltpu.VMEM_SHARED((npad, f), jnp.float32),
            pltpu.SemaphoreType.DMA,
            pltpu.SemaphoreType.DMA,
        ],
        compiler_params=cparams,
        interpret=interpret,
    )
    def edge_kernel(table_hbm, src_hbm, dst_hbm, zeros_hbm, out_hbm,
                    sidx, didx, rows, bnc, acc_sh, sem, sem2):
        c = lax.axis_index("c")
        s = lax.axis_index("s")
        wid = c * 16 + s

        nch = npt // 4
        for k in range(4):
            pltpu.sync_copy(zeros_hbm.at[pl.ds(s * npt + k * nch, nch)], bnc)
            pltpu.sync_copy(bnc, acc_sh.at[pl.ds(s * npt + k * nch, nch)])
        plsc.subcore_barrier()

        base = wid * rows_per_w

        def body(g, carry):
            r0 = base + g * (2 * group)
            pltpu.sync_copy(src_hbm.at[pl.ds(r0, 2 * group)], sidx)
            pltpu.sync_copy(dst_hbm.at[pl.ds(r0, 2 * group)], didx)
            ga = [pltpu.async_copy(table_hbm.at[sidx.at[j]], rows.at[j], sem)
                  for j in range(group)]
            for h in ga:
                h.wait()
            sa = [pltpu.async_copy(rows.at[j], acc_sh.at[didx.at[j]], sem2,
                                   add=True)
                  for j in range(group)]
            gb = [pltpu.async_copy(table_hbm.at[sidx.at[j]], rows.at[j], sem)
                  for j in range(group, 2 * group)]
            for h in gb:
                h.wait()
            sb = [pltpu.async_copy(rows.at[j], acc_sh.at[didx.at[j]], sem2,
                                   add=True)
                  for j in range(group, 2 * group)]
            for h in sa + sb:
                h.wait()
            return carry

        lax.fori_loop(0, rows_per_w // (2 * group), body, 0)
        plsc.subcore_barrier()
        for k in range(4):
            pltpu.sync_copy(acc_sh.at[pl.ds(s * npt + k * nch, nch)], bnc)
            pltpu.sync_copy(
                bnc, out_hbm.at[pl.ds(c * npad + s * npt + k * nch, nch)])

    return deg_kernel, edge_kernel


_deg_kernel, _edge_kernel = _build_kernels(_NPAD, _LANES, _GROUP, _ROWS_PER_W,
                                           _F)


def _gcn(deg_kernel, edge_kernel, n, npad, epad,
         x, edge_index, W1, b1, W2, b2, W3, b3, Wc, bc):
    e = edge_index.shape[1]
    lanes = _LANES
    pad_e = epad - e
    src = jnp.concatenate(
        [edge_index[0], jnp.full((pad_e,), n, jnp.int32)]).reshape(-1, lanes)
    dst = jnp.concatenate(
        [edge_index[1], jnp.full((pad_e,), n, jnp.int32)]).reshape(-1, lanes)

    zeros1 = jnp.zeros((npad,), jnp.float32)
    zerosF = jnp.zeros((npad, _F), jnp.float32)

    degp = deg_kernel(dst, zeros1)
    deg = degp[:n] + degp[npad:npad + n] + 1.0
    dis = lax.rsqrt(deg)

    def layer(h, W, b, fout):
        ts = (h @ W) * dis[:, None]
        tsp = jnp.zeros((npad, _F), jnp.float32).at[:n, :fout].set(ts)
        part = edge_kernel(tsp, src, dst, zerosF)
        agg = part[:n, :fout] + part[npad:npad + n, :fout] + ts
        return jnp.tanh(agg * dis[:, None] + b)

    h = layer(x, W1, b1, 4)
    h = layer(h, W2, b2, 4)
    h = layer(h, W3, b3, 3)
    out = h @ Wc + bc
    return (out, h)


def kernel(x, edge_index, W1, b1, W2, b2, W3, b3, Wc, bc):
    return _gcn(_deg_kernel, _edge_kernel, _N, _NPAD, _EPAD,
                x, edge_index, W1, b1, W2, b2, W3, b3, Wc, bc)
